# baseline (speedup 1.0000x reference)
import functools

import jax
import jax.numpy as jnp
from jax import lax
from jax.experimental import pallas as pl
from jax.experimental.pallas import tpu as pltpu
from jax.experimental.pallas import tpu_sc as plsc

NC = 2
NS = 16
NW = NC * NS
K = 128
NBUF = 2


def _row_map(i):
    return (i, jnp.int32(0))


def _zero_map(i):
    return (jnp.int32(0), jnp.int32(0))


def _zero_map1(i):
    return (jnp.int32(0),)


def _mesh():
    return plsc.VectorSubcoreMesh(core_axis_name="c", subcore_axis_name="s",
                                  num_cores=NC, num_subcores=NS)


def _fori(n, body):
    lax.fori_loop(jnp.int32(0), jnp.int32(n), body, jnp.int32(0))


def _make_degree_kernel(R, CH):
    @functools.partial(
        pl.kernel,
        out_type=(
            jax.ShapeDtypeStruct((NC, R), jnp.float32),
            jax.ShapeDtypeStruct((NC, R), jnp.float32),
        ),
        mesh=_mesh(),
        scratch_types=[
            pltpu.VMEM((CH, K), jnp.int32),
            pltpu.VMEM((CH, K), jnp.int32),
            pltpu.VMEM((K,), jnp.float32),
            pltpu.VMEM_SHARED((R,), jnp.float32),
            pltpu.VMEM_SHARED((R,), jnp.float32),
            pltpu.SemaphoreType.DMA,
            pltpu.SemaphoreType.DMA,
        ],
    )
    def deg_kernel(ni_hbm, ei_hbm, zvec_hbm, degv_out, dege_out,
                   nidx, eidx, ones_v, dv_acc, de_acc, sem_n, sem_e):
        c = lax.axis_index("c")
        s = lax.axis_index("s")
        wid = c * NS + s
        stripe = R // NS

        one = jnp.ones((16,), jnp.float32)

        def ones_body(i, _):
            ones_v[pl.ds(i * 16, 16)] = one
            return _

        _fori(K // 16, ones_body)
        pltpu.sync_copy(ni_hbm.at[pl.ds(wid * CH, CH)], nidx)
        pltpu.sync_copy(ei_hbm.at[pl.ds(wid * CH, CH)], eidx)
        pltpu.sync_copy(zvec_hbm, dv_acc.at[pl.ds(s * stripe, stripe)])
        pltpu.sync_copy(zvec_hbm, de_acc.at[pl.ds(s * stripe, stripe)])
        plsc.subcore_barrier()

        def fire(j, _):
            pltpu.async_copy(ones_v, dv_acc.at[nidx.at[j]], sem_n, add=True)
            pltpu.async_copy(ones_v, de_acc.at[eidx.at[j]], sem_e, add=True)
            return _

        _fori(CH, fire)

        def drain(j, _):
            pltpu.make_async_copy(ones_v, dv_acc.at[nidx.at[jnp.int32(0)]], sem_n).wait()
            pltpu.make_async_copy(ones_v, de_acc.at[eidx.at[jnp.int32(0)]], sem_e).wait()
            return _

        _fori(CH, drain)
        plsc.subcore_barrier()

        pltpu.sync_copy(dv_acc.at[pl.ds(s * stripe, stripe)],
                        degv_out.at[c, pl.ds(s * stripe, stripe)])
        pltpu.sync_copy(de_acc.at[pl.ds(s * stripe, stripe)],
                        dege_out.at[c, pl.ds(s * stripe, stripe)])

    return deg_kernel


def _make_row_pass_kernel(R, CH0, CH1, D, with_scalar):
    out_types = [jax.ShapeDtypeStruct((NC, R, D), jnp.float32)]
    scratch = [
        pltpu.VMEM((2, NBUF, K), jnp.int32),
        pltpu.VMEM((2, NBUF, K), jnp.int32),
        pltpu.VMEM((NBUF, K, D), jnp.float32),
        pltpu.VMEM_SHARED((R, D), jnp.float32),
        pltpu.SemaphoreType.DMA,
    ]
    scratch += [pltpu.SemaphoreType.DMA] * NBUF
    scratch += [pltpu.SemaphoreType.DMA] * NBUF
    if with_scalar:
        out_types.append(jax.ShapeDtypeStruct((NC, R), jnp.float32))
        scratch += [
            pltpu.VMEM((NBUF, K), jnp.float32),
            pltpu.VMEM_SHARED((R,), jnp.float32),
        ]
    G0 = CH0 // NBUF
    G1 = CH1 // NBUF
    assert G0 % 2 == 0 and G1 % 2 == 0

    def body(*refs):
        if with_scalar:
            (tab_hbm, stab_hbm, gi_hbm, si_hbm, zrows_hbm, zvec_hbm,
             rows_out, s_out, gidx, sidx, rowbuf, acc, isem,
             g0, g1, s0, s1, dvbuf, s_acc) = refs
        else:
            (tab_hbm, gi_hbm, si_hbm, zrows_hbm,
             rows_out, gidx, sidx, rowbuf, acc, isem,
             g0, g1, s0, s1) = refs
        gsem = [g0, g1]
        ssem = [s0, s1]
        c = lax.axis_index("c")
        s = lax.axis_index("s")
        stripe = R // NS
        i32 = jnp.int32
        groups_my = jnp.where(c == 0, i32(G0), i32(G1))
        off_rows = jnp.where(c == 0, s * CH0, i32(NS * CH0) + s * CH1)

        def fire_gathers(p):
            for b in range(NBUF):
                pltpu.async_copy(tab_hbm.at[gidx.at[p, i32(b)]],
                                 rowbuf.at[i32(b)], gsem[b])
                if with_scalar:
                    pltpu.async_copy(stab_hbm.at[gidx.at[p, i32(b)]],
                                     dvbuf.at[i32(b)], gsem[b])

        def wait_gather(b):
            pltpu.make_async_copy(tab_hbm.at[gidx.at[i32(0), i32(0)]],
                                  rowbuf.at[i32(b)], gsem[b]).wait()
            if with_scalar:
                pltpu.make_async_copy(stab_hbm.at[gidx.at[i32(0), i32(0)]],
                                      dvbuf.at[i32(b)], gsem[b]).wait()

        def fire_scatters(p, b):
            pltpu.async_copy(rowbuf.at[i32(b)], acc.at[sidx.at[p, i32(b)]],
                             ssem[b], add=True)
            if with_scalar:
                pltpu.async_copy(dvbuf.at[i32(b)], s_acc.at[sidx.at[p, i32(b)]],
                                 ssem[b], add=True)

        def wait_scatter(b):
            pltpu.make_async_copy(rowbuf.at[i32(b)],
                                  acc.at[sidx.at[i32(0), i32(0)]],
                                  ssem[b]).wait()
            if with_scalar:
                pltpu.make_async_copy(dvbuf.at[i32(b)],
                                      s_acc.at[sidx.at[i32(0), i32(0)]],
                                      ssem[b]).wait()

        def fire_idx_load(grp_idx, p):
            base = off_rows + grp_idx * NBUF
            pltpu.async_copy(gi_hbm.at[pl.ds(base, NBUF)], gidx.at[p], isem)
            pltpu.async_copy(si_hbm.at[pl.ds(base, NBUF)], sidx.at[p], isem)

        def wait_idx_load():
            pltpu.make_async_copy(gi_hbm.at[pl.ds(0, NBUF)],
                                  gidx.at[i32(0)], isem).wait()
            pltpu.make_async_copy(si_hbm.at[pl.ds(0, NBUF)],
                                  sidx.at[i32(0)], isem).wait()

        pltpu.sync_copy(gi_hbm.at[pl.ds(off_rows, NBUF)], gidx.at[i32(0)])
        pltpu.sync_copy(si_hbm.at[pl.ds(off_rows, NBUF)], sidx.at[i32(0)])
        pltpu.sync_copy(zrows_hbm, acc.at[pl.ds(s * stripe, stripe)])
        if with_scalar:
            pltpu.sync_copy(zvec_hbm, s_acc.at[pl.ds(s * stripe, stripe)])
        plsc.subcore_barrier()

        fire_idx_load(jnp.int32(1), i32(1))
        fire_gathers(i32(0))

        def grp(g, _):
            p = lax.rem(g, jnp.int32(2))
            pn = lax.rem(g + 1, jnp.int32(2))
            for b in range(NBUF):
                wait_gather(b)
                fire_scatters(p, b)
            wait_idx_load()
            for b in range(NBUF):
                wait_scatter(b)
            fire_gathers(pn)
            gnext = jnp.minimum(g + 2, groups_my - 1)
            fire_idx_load(gnext, p)
            return _

        lax.fori_loop(jnp.int32(0), groups_my - 1, grp, jnp.int32(0))

        pe = jnp.int32(1)
        for b in range(NBUF):
            wait_gather(b)
            fire_scatters(pe, b)
        wait_idx_load()
        for b in range(NBUF):
            wait_scatter(b)
        plsc.subcore_barrier()

        pltpu.sync_copy(acc.at[pl.ds(s * stripe, stripe)],
                        rows_out.at[c, pl.ds(s * stripe, stripe)])
        if with_scalar:
            pltpu.sync_copy(s_acc.at[pl.ds(s * stripe, stripe)],
                            s_out.at[c, pl.ds(s * stripe, stripe)])

    return pl.kernel(
        body,
        out_type=tuple(out_types) if with_scalar else out_types[0],
        mesh=_mesh(),
        scratch_types=scratch,
    )


def _x_matmul(xp, W):
    R = xp.shape[0]
    BLK = 512

    def body(x_r, W_r, out_r):
        out_r[...] = jnp.dot(x_r[...], W_r[...],
                             preferred_element_type=jnp.float32)

    return pl.pallas_call(
        body,
        grid=(R // BLK,),
        in_specs=[
            pl.BlockSpec((BLK, 128), _row_map),
            pl.BlockSpec((128, 128), _zero_map),
        ],
        out_specs=pl.BlockSpec((BLK, 128), _row_map),
        out_shape=jax.ShapeDtypeStruct((R, 128), jnp.float32),
    )(xp, W)


def _edge_scale(he0, he1, dvs0, dvs1, de0, de1, b):
    R = he0.shape[0]
    BLK = 512

    def body(he0_r, he1_r, dvs0_r, dvs1_r, de0_r, de1_r, b_r, out_r):
        dege = jnp.maximum(de0_r[...] + de1_r[...], 1.0)
        hesum = he0_r[...] + he1_r[...]
        dvs = dvs0_r[...] + dvs1_r[...]
        det = jnp.maximum(dvs / dege, 1.0)
        he = hesum * (1.0 / dege)[:, None] + b_r[...][None, :]
        out_r[...] = he * lax.rsqrt(det)[:, None]

    return pl.pallas_call(
        body,
        grid=(R // BLK,),
        in_specs=[
            pl.BlockSpec((BLK, 128), _row_map),
            pl.BlockSpec((BLK, 128), _row_map),
            pl.BlockSpec((BLK,), lambda i: (i,)),
            pl.BlockSpec((BLK,), lambda i: (i,)),
            pl.BlockSpec((BLK,), lambda i: (i,)),
            pl.BlockSpec((BLK,), lambda i: (i,)),
            pl.BlockSpec((128,), _zero_map1),
        ],
        out_specs=pl.BlockSpec((BLK, 128), _row_map),
        out_shape=jax.ShapeDtypeStruct((R, 128), jnp.float32),
    )(he0, he1, dvs0, dvs1, de0, de1, b)


def _readout(agg0, agg1, dvt, Wr, br):
    R = agg0.shape[0]
    BLK = 512

    def body(a0_r, a1_r, dv_r, wr_r, br_r, out_r):
        agg = a0_r[...] + a1_r[...]
        dv = jnp.maximum(dv_r[...], 1.0)
        h = jnp.maximum(agg * lax.rsqrt(dv)[:, None], 0.0)
        y = jnp.sum(h * wr_r[...], axis=-1) + br_r[0]
        out_r[...] = y

    return pl.pallas_call(
        body,
        grid=(R // BLK,),
        in_specs=[
            pl.BlockSpec((BLK, 128), _row_map),
            pl.BlockSpec((BLK, 128), _row_map),
            pl.BlockSpec((BLK,), lambda i: (i,)),
            pl.BlockSpec((1, 128), _zero_map),
            pl.BlockSpec((1,), _zero_map1, memory_space=pltpu.SMEM),
        ],
        out_specs=pl.BlockSpec((BLK,), lambda i: (i,)),
        out_shape=jax.ShapeDtypeStruct((R,), jnp.float32),
    )(agg0, agg1, dvt, Wr, br)


def kernel(x, hyperedge_index, W_conv, b_conv, W_read, b_read):
    N, D = x.shape
    NNZ = hyperedge_index.shape[1]
    out_dtype = jnp.result_type(x.dtype, W_conv.dtype, W_read.dtype)
    x = x.astype(jnp.float32)
    W_conv = W_conv.astype(jnp.float32)
    b_conv = b_conv.astype(jnp.float32)
    W_read = W_read.astype(jnp.float32)
    b_read = b_read.astype(jnp.float32)

    R = ((N + 1 + (NS * K) - 1) // (NS * K)) * (NS * K)
    CHP = -(-NNZ // (NS * K))
    CHP = ((CHP + 2 * NBUF - 1) // (2 * NBUF)) * (2 * NBUF)
    CH0 = max(4, int(round(CHP * 0.225 / 4)) * 4)
    CH1 = CHP - CH0
    CH = CHP // 2
    total = CHP * NS * K
    pad = total - NNZ
    trash = N

    ni = hyperedge_index[0].astype(jnp.int32)
    ei = hyperedge_index[1].astype(jnp.int32)
    zpad = jnp.zeros((pad,), jnp.int32)
    tpad = jnp.full((pad,), trash, jnp.int32)
    ni_g = jnp.concatenate([ni, zpad]).reshape(NW * CH, K)
    ni_s = jnp.concatenate([ni, tpad]).reshape(NW * CH, K)
    ei_g = jnp.concatenate([ei, zpad]).reshape(NW * CH, K)
    ei_s = jnp.concatenate([ei, tpad]).reshape(NW * CH, K)

    stripe = R // NS
    zrows = jnp.zeros((stripe, D), jnp.float32)
    zvec = jnp.zeros((stripe,), jnp.float32)
    xp = jnp.concatenate([x, jnp.zeros((R - N, D), jnp.float32)])

    xw = _x_matmul(xp, W_conv)

    deg = _make_degree_kernel(R, CH)
    degv_part, dege_part = deg(ni_s, ei_s, zvec)
    dv_tab = degv_part[0] + degv_part[1]

    rowpass = _make_row_pass_kernel(R, CH0, CH1, D, with_scalar=True)
    he_part, dvs_part = rowpass(xw, dv_tab, ni_g, ei_s, zrows, zvec)

    he_s = _edge_scale(he_part[0], he_part[1], dvs_part[0], dvs_part[1],
                       dege_part[0], dege_part[1], b_conv)

    aggpass = _make_row_pass_kernel(R, CH0, CH1, D, with_scalar=False)
    agg_part = aggpass(he_s, ei_g, ni_s, zrows)

    y = _readout(agg_part[0], agg_part[1], dv_tab, W_read, b_read)
    return y[:N].astype(out_dtype)


# speedup vs baseline: 1.2721x; 1.2721x over previous
import functools

import jax
import jax.numpy as jnp
from jax import lax
from jax.experimental import pallas as pl
from jax.experimental.pallas import tpu as pltpu
from jax.experimental.pallas import tpu_sc as plsc

NC = 2
NS = 16
NW = NC * NS
K = 128
NBUF = 2


def _row_map(i):
    return (i, jnp.int32(0))


def _zero_map(i):
    return (jnp.int32(0), jnp.int32(0))


def _zero_map1(i):
    return (jnp.int32(0),)


def _mesh():
    return plsc.VectorSubcoreMesh(core_axis_name="c", subcore_axis_name="s",
                                  num_cores=NC, num_subcores=NS)


def _fori(n, body):
    lax.fori_loop(jnp.int32(0), jnp.int32(n), body, jnp.int32(0))


def _make_degree_kernel(R, CH):
    @functools.partial(
        pl.kernel,
        out_type=(
            jax.ShapeDtypeStruct((NC, R), jnp.float32),
            jax.ShapeDtypeStruct((NC, R), jnp.float32),
        ),
        mesh=_mesh(),
        scratch_types=[
            pltpu.VMEM((CH, K), jnp.int32),
            pltpu.VMEM((CH, K), jnp.int32),
            pltpu.VMEM((K,), jnp.float32),
            pltpu.VMEM_SHARED((R,), jnp.float32),
            pltpu.VMEM_SHARED((R,), jnp.float32),
            pltpu.SemaphoreType.DMA,
            pltpu.SemaphoreType.DMA,
        ],
    )
    def deg_kernel(ni_hbm, ei_hbm, zvec_hbm, degv_out, dege_out,
                   nidx, eidx, ones_v, dv_acc, de_acc, sem_n, sem_e):
        c = lax.axis_index("c")
        s = lax.axis_index("s")
        wid = c * NS + s
        stripe = R // NS

        one = jnp.ones((16,), jnp.float32)

        def ones_body(i, _):
            ones_v[pl.ds(i * 16, 16)] = one
            return _

        _fori(K // 16, ones_body)
        pltpu.sync_copy(ni_hbm.at[pl.ds(wid * CH, CH)], nidx)
        pltpu.sync_copy(ei_hbm.at[pl.ds(wid * CH, CH)], eidx)
        pltpu.sync_copy(zvec_hbm, dv_acc.at[pl.ds(s * stripe, stripe)])
        pltpu.sync_copy(zvec_hbm, de_acc.at[pl.ds(s * stripe, stripe)])
        plsc.subcore_barrier()

        def fire(j, _):
            pltpu.async_copy(ones_v, dv_acc.at[nidx.at[j]], sem_n, add=True)
            pltpu.async_copy(ones_v, de_acc.at[eidx.at[j]], sem_e, add=True)
            return _

        _fori(CH, fire)

        def drain(j, _):
            pltpu.make_async_copy(ones_v, dv_acc.at[nidx.at[jnp.int32(0)]], sem_n).wait()
            pltpu.make_async_copy(ones_v, de_acc.at[eidx.at[jnp.int32(0)]], sem_e).wait()
            return _

        _fori(CH, drain)
        plsc.subcore_barrier()

        pltpu.sync_copy(dv_acc.at[pl.ds(s * stripe, stripe)],
                        degv_out.at[c, pl.ds(s * stripe, stripe)])
        pltpu.sync_copy(de_acc.at[pl.ds(s * stripe, stripe)],
                        dege_out.at[c, pl.ds(s * stripe, stripe)])

    return deg_kernel


def _make_row_pass_kernel(R, CH0, CH1, D, with_scalar):
    out_types = [jax.ShapeDtypeStruct((NC, R, D), jnp.float32)]
    scratch = [
        pltpu.VMEM((2, NBUF, K), jnp.int32),
        pltpu.VMEM((2, NBUF, K), jnp.int32),
        pltpu.VMEM((NBUF, K, D), jnp.float32),
        pltpu.VMEM_SHARED((R, D), jnp.float32),
        pltpu.SemaphoreType.DMA,
    ]
    scratch += [pltpu.SemaphoreType.DMA] * NBUF
    scratch += [pltpu.SemaphoreType.DMA] * NBUF
    if with_scalar:
        out_types.append(jax.ShapeDtypeStruct((NC, R), jnp.float32))
        scratch += [
            pltpu.VMEM((NBUF, K), jnp.float32),
            pltpu.VMEM_SHARED((R,), jnp.float32),
        ]
    G0 = CH0 // NBUF
    G1 = CH1 // NBUF
    assert G0 % 2 == 0 and G1 % 2 == 0

    def body(*refs):
        if with_scalar:
            (tab_hbm, stab_hbm, gi_hbm, si_hbm, zrows_hbm, zvec_hbm,
             rows_out, s_out, gidx, sidx, rowbuf, acc, isem,
             g0, g1, s0, s1, dvbuf, s_acc) = refs
        else:
            (tab_hbm, gi_hbm, si_hbm, zrows_hbm,
             rows_out, gidx, sidx, rowbuf, acc, isem,
             g0, g1, s0, s1) = refs
        gsem = [g0, g1]
        ssem = [s0, s1]
        c = lax.axis_index("c")
        s = lax.axis_index("s")
        stripe = R // NS
        i32 = jnp.int32
        groups_my = jnp.where(c == 0, i32(G0), i32(G1))
        off_rows = jnp.where(c == 0, s * CH0, i32(NS * CH0) + s * CH1)

        def fire_gathers(p):
            for b in range(NBUF):
                pltpu.async_copy(tab_hbm.at[gidx.at[p, i32(b)]],
                                 rowbuf.at[i32(b)], gsem[b])
                if with_scalar:
                    pltpu.async_copy(stab_hbm.at[gidx.at[p, i32(b)]],
                                     dvbuf.at[i32(b)], gsem[b])

        def wait_gather(b):
            pltpu.make_async_copy(tab_hbm.at[gidx.at[i32(0), i32(0)]],
                                  rowbuf.at[i32(b)], gsem[b]).wait()
            if with_scalar:
                pltpu.make_async_copy(stab_hbm.at[gidx.at[i32(0), i32(0)]],
                                      dvbuf.at[i32(b)], gsem[b]).wait()

        def fire_scatters(p, b):
            pltpu.async_copy(rowbuf.at[i32(b)], acc.at[sidx.at[p, i32(b)]],
                             ssem[b], add=True)
            if with_scalar:
                pltpu.async_copy(dvbuf.at[i32(b)], s_acc.at[sidx.at[p, i32(b)]],
                                 ssem[b], add=True)

        def wait_scatter(b):
            pltpu.make_async_copy(rowbuf.at[i32(b)],
                                  acc.at[sidx.at[i32(0), i32(0)]],
                                  ssem[b]).wait()
            if with_scalar:
                pltpu.make_async_copy(dvbuf.at[i32(b)],
                                      s_acc.at[sidx.at[i32(0), i32(0)]],
                                      ssem[b]).wait()

        def fire_idx_load(grp_idx, p):
            base = off_rows + grp_idx * NBUF
            pltpu.async_copy(gi_hbm.at[pl.ds(base, NBUF)], gidx.at[p], isem)
            pltpu.async_copy(si_hbm.at[pl.ds(base, NBUF)], sidx.at[p], isem)

        def wait_idx_load():
            pltpu.make_async_copy(gi_hbm.at[pl.ds(0, NBUF)],
                                  gidx.at[i32(0)], isem).wait()
            pltpu.make_async_copy(si_hbm.at[pl.ds(0, NBUF)],
                                  sidx.at[i32(0)], isem).wait()

        pltpu.sync_copy(gi_hbm.at[pl.ds(off_rows, NBUF)], gidx.at[i32(0)])
        pltpu.sync_copy(si_hbm.at[pl.ds(off_rows, NBUF)], sidx.at[i32(0)])
        pltpu.sync_copy(zrows_hbm, acc.at[pl.ds(s * stripe, stripe)])
        if with_scalar:
            pltpu.sync_copy(zvec_hbm, s_acc.at[pl.ds(s * stripe, stripe)])
        plsc.subcore_barrier()

        fire_idx_load(jnp.int32(1), i32(1))
        fire_gathers(i32(0))

        def grp(g, _):
            p = lax.rem(g, jnp.int32(2))
            pn = lax.rem(g + 1, jnp.int32(2))
            for b in range(NBUF):
                wait_gather(b)
                fire_scatters(p, b)
            wait_idx_load()
            for b in range(NBUF):
                wait_scatter(b)
            fire_gathers(pn)
            gnext = jnp.minimum(g + 2, groups_my - 1)
            fire_idx_load(gnext, p)
            return _

        lax.fori_loop(jnp.int32(0), groups_my - 1, grp, jnp.int32(0))

        pe = jnp.int32(1)
        for b in range(NBUF):
            wait_gather(b)
            fire_scatters(pe, b)
        wait_idx_load()
        for b in range(NBUF):
            wait_scatter(b)
        plsc.subcore_barrier()

        pltpu.sync_copy(acc.at[pl.ds(s * stripe, stripe)],
                        rows_out.at[c, pl.ds(s * stripe, stripe)])
        if with_scalar:
            pltpu.sync_copy(s_acc.at[pl.ds(s * stripe, stripe)],
                            s_out.at[c, pl.ds(s * stripe, stripe)])

    return pl.kernel(
        body,
        out_type=tuple(out_types) if with_scalar else out_types[0],
        mesh=_mesh(),
        scratch_types=scratch,
    )


def _x_matmul(xp, W):
    R = xp.shape[0]
    BLK = 512

    def body(x_r, W_r, out_r):
        out_r[...] = jnp.dot(x_r[...], W_r[...],
                             preferred_element_type=jnp.float32)

    return pl.pallas_call(
        body,
        grid=(R // BLK,),
        in_specs=[
            pl.BlockSpec((BLK, 128), _row_map),
            pl.BlockSpec((128, 128), _zero_map),
        ],
        out_specs=pl.BlockSpec((BLK, 128), _row_map),
        out_shape=jax.ShapeDtypeStruct((R, 128), jnp.float32),
    )(xp, W)


def _edge_scale(he0, he1, dvs0, dvs1, de0, de1, b):
    R = he0.shape[0]
    BLK = 512

    def body(he0_r, he1_r, dvs0_r, dvs1_r, de0_r, de1_r, b_r, out_r):
        dege = jnp.maximum(de0_r[...] + de1_r[...], 1.0)
        hesum = he0_r[...] + he1_r[...]
        dvs = dvs0_r[...] + dvs1_r[...]
        det = jnp.maximum(dvs / dege, 1.0)
        he = hesum * (1.0 / dege)[:, None] + b_r[...][None, :]
        out_r[...] = he * lax.rsqrt(det)[:, None]

    return pl.pallas_call(
        body,
        grid=(R // BLK,),
        in_specs=[
            pl.BlockSpec((BLK, 128), _row_map),
            pl.BlockSpec((BLK, 128), _row_map),
            pl.BlockSpec((BLK,), lambda i: (i,)),
            pl.BlockSpec((BLK,), lambda i: (i,)),
            pl.BlockSpec((BLK,), lambda i: (i,)),
            pl.BlockSpec((BLK,), lambda i: (i,)),
            pl.BlockSpec((128,), _zero_map1),
        ],
        out_specs=pl.BlockSpec((BLK, 128), _row_map),
        out_shape=jax.ShapeDtypeStruct((R, 128), jnp.float32),
    )(he0, he1, dvs0, dvs1, de0, de1, b)


def _readout(agg0, agg1, dvt, Wr, br):
    R = agg0.shape[0]
    BLK = 512

    def body(a0_r, a1_r, dv_r, wr_r, br_r, out_r):
        agg = a0_r[...] + a1_r[...]
        dv = jnp.maximum(dv_r[...], 1.0)
        h = jnp.maximum(agg * lax.rsqrt(dv)[:, None], 0.0)
        y = jnp.sum(h * wr_r[...], axis=-1) + br_r[0]
        out_r[...] = y

    return pl.pallas_call(
        body,
        grid=(R // BLK,),
        in_specs=[
            pl.BlockSpec((BLK, 128), _row_map),
            pl.BlockSpec((BLK, 128), _row_map),
            pl.BlockSpec((BLK,), lambda i: (i,)),
            pl.BlockSpec((1, 128), _zero_map),
            pl.BlockSpec((1,), _zero_map1, memory_space=pltpu.SMEM),
        ],
        out_specs=pl.BlockSpec((BLK,), lambda i: (i,)),
        out_shape=jax.ShapeDtypeStruct((R,), jnp.float32),
    )(agg0, agg1, dvt, Wr, br)


def kernel(x, hyperedge_index, W_conv, b_conv, W_read, b_read):
    N, D = x.shape
    NNZ = hyperedge_index.shape[1]
    out_dtype = jnp.result_type(x.dtype, W_conv.dtype, W_read.dtype)
    x = x.astype(jnp.float32)
    W_conv = W_conv.astype(jnp.float32)
    b_conv = b_conv.astype(jnp.float32)
    W_read = W_read.astype(jnp.float32)
    b_read = b_read.astype(jnp.float32)

    R = ((N + 1 + (NS * K) - 1) // (NS * K)) * (NS * K)
    CHP = -(-NNZ // (NS * K))
    CHP = ((CHP + 2 * NBUF - 1) // (2 * NBUF)) * (2 * NBUF)
    CH1 = max(4, int(round(CHP * 0.225 / 4)) * 4)
    CH0 = CHP - CH1
    CH = CHP // 2
    total = CHP * NS * K
    pad = total - NNZ
    trash = N

    ni = hyperedge_index[0].astype(jnp.int32)
    ei = hyperedge_index[1].astype(jnp.int32)
    zpad = jnp.zeros((pad,), jnp.int32)
    tpad = jnp.full((pad,), trash, jnp.int32)
    ni_g = jnp.concatenate([ni, zpad]).reshape(NW * CH, K)
    ni_s = jnp.concatenate([ni, tpad]).reshape(NW * CH, K)
    ei_g = jnp.concatenate([ei, zpad]).reshape(NW * CH, K)
    ei_s = jnp.concatenate([ei, tpad]).reshape(NW * CH, K)

    stripe = R // NS
    zrows = jnp.zeros((stripe, D), jnp.float32)
    zvec = jnp.zeros((stripe,), jnp.float32)
    xp = jnp.concatenate([x, jnp.zeros((R - N, D), jnp.float32)])

    xw = _x_matmul(xp, W_conv)

    deg = _make_degree_kernel(R, CH)
    degv_part, dege_part = deg(ni_s, ei_s, zvec)
    dv_tab = degv_part[0] + degv_part[1]

    rowpass = _make_row_pass_kernel(R, CH0, CH1, D, with_scalar=True)
    he_part, dvs_part = rowpass(xw, dv_tab, ni_g, ei_s, zrows, zvec)

    he_s = _edge_scale(he_part[0], he_part[1], dvs_part[0], dvs_part[1],
                       dege_part[0], dege_part[1], b_conv)

    aggpass = _make_row_pass_kernel(R, CH0, CH1, D, with_scalar=False)
    agg_part = aggpass(he_s, ei_g, ni_s, zrows)

    y = _readout(agg_part[0], agg_part[1], dv_tab, W_read, b_read)
    return y[:N].astype(out_dtype)
